# baseline (speedup 1.0000x reference)
import jax
import jax.numpy as jnp
from jax import lax
from jax.experimental import pallas as pl
from jax.experimental.pallas import tpu as pltpu
from jax.experimental.pallas import tpu_sc as plsc

import functools

_NC = 2
_NS = 16
_L = 16


@functools.partial(jax.jit, static_argnames=("batch", "seq_len", "half"))
def _gather_last(seq_rows, lengths, *, batch, seq_len, half):
    nrows = 2 * batch
    mesh = plsc.VectorSubcoreMesh(
        core_axis_name="c", subcore_axis_name="s", num_cores=_NC, num_subcores=_NS
    )

    @functools.partial(
        pl.kernel,
        out_type=jax.ShapeDtypeStruct((nrows, half), jnp.float32),
        mesh=mesh,
        scratch_types=[
            pltpu.VMEM((_L,), jnp.int32),
            pltpu.VMEM((_L, half), jnp.float32),
            pltpu.SemaphoreType.DMA,
        ],
    )
    def k(seq_hbm, len_hbm, out_hbm, len_v, rows_v, sem):
        wid = lax.axis_index("s") * _NC + lax.axis_index("c")
        b = wid // 2
        is_bw = wid % 2

        pltpu.sync_copy(len_hbm, len_v)
        lv = len_v[...]
        biota = lax.iota(jnp.int32, _L)
        fw_rows = biota * (2 * seq_len) + 2 * (lv - 1)
        bw_rows = biota * (2 * seq_len) + 1
        rows = jnp.where(is_bw == 1, bw_rows, fw_rows)

        pltpu.async_copy(seq_hbm.at[rows], rows_v, sem).wait()
        pltpu.sync_copy(rows_v.at[pl.ds(b, 1)], out_hbm.at[pl.ds(wid, 1)])

    return k(seq_rows, lengths)


def kernel(sequences, lengths):
    batch, seq_len, hidden_x_dirs = sequences.shape
    half = hidden_x_dirs // 2
    seq_rows = sequences.reshape(batch * seq_len * 2, half)
    out = _gather_last(
        seq_rows, lengths.astype(jnp.int32), batch=batch, seq_len=seq_len, half=half
    )
    return out.reshape(batch, hidden_x_dirs)


# speedup vs baseline: 12.9768x; 12.9768x over previous
import jax
import jax.numpy as jnp
from jax import lax
from jax.experimental import pallas as pl
from jax.experimental.pallas import tpu as pltpu
from jax.experimental.pallas import tpu_sc as plsc

import functools

_NC = 2
_NS = 16
_L = 16


@functools.partial(jax.jit, static_argnames=("batch", "seq_len", "hidden"))
def _gather_last(seq_rows, lengths, *, batch, seq_len, hidden):
    half = hidden // 2
    mesh = plsc.VectorSubcoreMesh(
        core_axis_name="c", subcore_axis_name="s", num_cores=_NC, num_subcores=_NS
    )

    @functools.partial(
        pl.kernel,
        out_type=jax.ShapeDtypeStruct((batch, hidden), jnp.float32),
        mesh=mesh,
        scratch_types=[
            pltpu.VMEM((_L,), jnp.int32),
            pltpu.VMEM((_L, hidden), jnp.float32),
            pltpu.SemaphoreType.DMA,
        ],
    )
    def k(seq_hbm, len_hbm, out_hbm, len_v, rows_v, sem):
        core = lax.axis_index("c")
        sub = lax.axis_index("s")
        biota = lax.iota(jnp.int32, _L)

        @pl.when(jnp.logical_and(core == 0, sub == 0))
        def _fw():
            pltpu.sync_copy(len_hbm, len_v)
            rows = biota * seq_len + (len_v[...] - 1)
            pltpu.async_copy(seq_hbm.at[rows], rows_v, sem).wait()
            pltpu.sync_copy(
                rows_v.at[:, pl.ds(0, half)], out_hbm.at[:, pl.ds(0, half)]
            )

        @pl.when(jnp.logical_and(core == 1, sub == 0))
        def _bw():
            rows = biota * seq_len
            pltpu.async_copy(seq_hbm.at[rows], rows_v, sem).wait()
            pltpu.sync_copy(
                rows_v.at[:, pl.ds(half, half)], out_hbm.at[:, pl.ds(half, half)]
            )

    return k(seq_rows, lengths)


def kernel(sequences, lengths):
    batch, seq_len, hidden_x_dirs = sequences.shape
    seq_rows = sequences.reshape(batch * seq_len, hidden_x_dirs)
    return _gather_last(
        seq_rows,
        lengths.astype(jnp.int32),
        batch=batch,
        seq_len=seq_len,
        hidden=hidden_x_dirs,
    )


# speedup vs baseline: 13.0786x; 1.0078x over previous
import jax
import jax.numpy as jnp
from jax import lax
from jax.experimental import pallas as pl
from jax.experimental.pallas import tpu as pltpu
from jax.experimental.pallas import tpu_sc as plsc

import functools

_NC = 2


@functools.partial(jax.jit, static_argnames=("batch", "seq_len", "hidden"))
def _gather_last(seq_rows, lengths, *, batch, seq_len, hidden):
    half = hidden // 2
    mesh = plsc.ScalarSubcoreMesh(axis_name="c", num_cores=_NC)

    @functools.partial(
        pl.kernel,
        out_type=jax.ShapeDtypeStruct((batch, hidden), jnp.float32),
        mesh=mesh,
        scratch_types=[
            pltpu.SMEM((batch,), jnp.int32),
            pltpu.SemaphoreType.DMA,
        ],
    )
    def k(seq_hbm, len_hbm, out_hbm, len_sm, sem):
        core = lax.axis_index("c")

        @pl.when(core == 0)
        def _fw():
            pltpu.sync_copy(len_hbm, len_sm)
            copies = []
            for b in range(batch):
                row = b * seq_len + len_sm[b] - 1
                copies.append(
                    pltpu.async_copy(
                        seq_hbm.at[pl.ds(row, 1), pl.ds(0, half)],
                        out_hbm.at[pl.ds(b, 1), pl.ds(0, half)],
                        sem,
                    )
                )
            for c in copies:
                c.wait()

        @pl.when(core == 1)
        def _bw():
            copies = []
            for b in range(batch):
                copies.append(
                    pltpu.async_copy(
                        seq_hbm.at[pl.ds(b * seq_len, 1), pl.ds(half, half)],
                        out_hbm.at[pl.ds(b, 1), pl.ds(half, half)],
                        sem,
                    )
                )
            for c in copies:
                c.wait()

    return k(seq_rows, lengths)


def kernel(sequences, lengths):
    batch, seq_len, hidden_x_dirs = sequences.shape
    seq_rows = sequences.reshape(batch * seq_len, hidden_x_dirs)
    return _gather_last(
        seq_rows,
        lengths.astype(jnp.int32),
        batch=batch,
        seq_len=seq_len,
        hidden=hidden_x_dirs,
    )


# speedup vs baseline: 13.4731x; 1.0302x over previous
import jax
import jax.numpy as jnp
from jax import lax
from jax.experimental import pallas as pl
from jax.experimental.pallas import tpu as pltpu
from jax.experimental.pallas import tpu_sc as plsc

import functools

_NC = 2


@functools.partial(jax.jit, static_argnames=("batch", "seq_len", "hidden"))
def _gather_last(seq_rows, lengths, *, batch, seq_len, hidden):
    half = hidden // 2
    mesh = plsc.ScalarSubcoreMesh(axis_name="c", num_cores=1)

    @functools.partial(
        pl.kernel,
        out_type=jax.ShapeDtypeStruct((batch, hidden), jnp.float32),
        mesh=mesh,
        scratch_types=[
            pltpu.SMEM((batch,), jnp.int32),
            pltpu.SemaphoreType.DMA,
        ],
    )
    def k(seq_hbm, len_hbm, out_hbm, len_sm, sem):
        pltpu.sync_copy(len_hbm, len_sm)
        copies = []
        for b in range(batch):
            row = b * seq_len + len_sm[b] - 1
            copies.append(
                pltpu.async_copy(
                    seq_hbm.at[pl.ds(row, 1), pl.ds(0, half)],
                    out_hbm.at[pl.ds(b, 1), pl.ds(0, half)],
                    sem,
                )
            )
            copies.append(
                pltpu.async_copy(
                    seq_hbm.at[pl.ds(b * seq_len, 1), pl.ds(half, half)],
                    out_hbm.at[pl.ds(b, 1), pl.ds(half, half)],
                    sem,
                )
            )
        for c in copies:
            c.wait()

    return k(seq_rows, lengths)


def kernel(sequences, lengths):
    batch, seq_len, hidden_x_dirs = sequences.shape
    seq_rows = sequences.reshape(batch * seq_len, hidden_x_dirs)
    return _gather_last(
        seq_rows,
        lengths.astype(jnp.int32),
        batch=batch,
        seq_len=seq_len,
        hidden=hidden_x_dirs,
    )
